# baseline (speedup 1.0000x reference)
import functools

import jax
import jax.numpy as jnp
from jax import lax
from jax.experimental import pallas as pl
from jax.experimental.pallas import tpu as pltpu
from jax.experimental.pallas import tpu_sc as plsc

_SLOPE = 0.2
_EPS = 1e-5


_NW = 32
_SUB = 128
_NSUB = 8


def _sc_gather(table, idx2):
    dt = table.dtype
    n_rows_idx, lw = idx2.shape
    r_total = n_rows_idx * lw
    ch = _SUB * _NSUB
    per_w = r_total // _NW
    n_it = per_w // ch

    mesh = plsc.VectorSubcoreMesh(core_axis_name="c", subcore_axis_name="s")

    @functools.partial(
        pl.kernel,
        mesh=mesh,
        out_type=jax.ShapeDtypeStruct((r_total, 32), dt),
        scratch_types=[
            pltpu.VMEM((_NSUB, _SUB), jnp.int32),
            pltpu.VMEM((ch, 32), dt),
            pltpu.SemaphoreType.DMA,
        ],
        compiler_params=pltpu.CompilerParams(use_tc_tiling_on_sc=False),
    )
    def gather_kernel(t_hbm, idx_hbm, o_hbm, idx_v, r_v, sem):
        wid = lax.axis_index("s") * 2 + lax.axis_index("c")
        base_w = wid * per_w

        def body(it, carry):
            base = pl.multiple_of(base_w + it * ch, ch)
            idx_row = pl.multiple_of(base // lw, _NSUB)
            pltpu.sync_copy(idx_hbm.at[pl.ds(idx_row, _NSUB)], idx_v)
            copies = []
            for j in range(_NSUB):
                copies.append(pltpu.async_copy(
                    t_hbm.at[idx_v.at[j]],
                    r_v.at[pl.ds(j * _SUB, _SUB)], sem))
            for cp in copies:
                cp.wait()
            pltpu.sync_copy(r_v, o_hbm.at[pl.ds(base, ch)])
            return carry

        lax.fori_loop(0, n_it, body, 0)

    return gather_kernel(table, idx2)


_RB = 256


def _lrelu(v):
    return jnp.where(v >= 0, v, _SLOPE * v)


def _feature(xg, qg, xc, ca, qbdt, uptt):
    rb = xc.shape[0]
    u_pt = jnp.dot(ca, uptt, preferred_element_type=jnp.float32)
    cent = u_pt - xc
    cent = jnp.concatenate([cent, cent, cent, cent], axis=1)
    cent = jnp.broadcast_to(cent[:, None, :], (rb, 8, 128))
    cent = cent.reshape(rb * 8, 128)
    qpart = jnp.dot(qg, qbdt, preferred_element_type=jnp.float32)
    return _lrelu(xg.astype(jnp.float32) + qpart + cent)


def _moments(v):
    mm = lax.dot_general(v, v, (((0,), (0,)), ((), ())),
                         preferred_element_type=jnp.float32)
    m = v.shape[0]
    ss = jnp.sum(v.reshape(8, m // 8, 128), axis=1)
    return mm, ss


def _acc(ref, val, step):
    @pl.when(step == 0)
    def _():
        ref[...] = val

    @pl.when(step != 0)
    def _():
        ref[...] += val


def _p1_body(qg_ref, ca_ref, m_ref, s_ref):
    i = pl.program_id(0)
    rb = ca_ref.shape[0]
    ca = ca_ref[...]
    cat = jnp.concatenate([ca, ca, ca, ca], axis=1)
    cat = jnp.broadcast_to(cat[:, None, :], (rb, 8, 128)).reshape(rb * 8, 128)
    w = qg_ref[...] + cat
    mm, ss = _moments(w)
    _acc(m_ref, mm, i)
    _acc(s_ref, ss, i)


def _p2_body(xg_ref, qg_ref, xc_ref, ca_ref, qbdt_ref, uptt_ref,
             m_ref, s_ref, f_ref):
    i = pl.program_id(0)
    f = _feature(xg_ref[...], qg_ref[...], xc_ref[...], ca_ref[...],
                 qbdt_ref[...], uptt_ref[...])
    f_ref[...] = f.astype(jnp.bfloat16)
    mm, ss = _moments(f)
    _acc(m_ref, mm, i)
    _acc(s_ref, ss, i)


def _p3_body(f_ref, a1_ref, c1_ref, m_ref, s_ref, h_ref):
    i = pl.program_id(0)
    f = f_ref[...].astype(jnp.float32)
    h1 = _lrelu(jnp.dot(f, a1_ref[...], preferred_element_type=jnp.float32)
                + c1_ref[...][:1])
    h_ref[...] = h1.astype(jnp.bfloat16)
    mm, ss = _moments(h1)
    _acc(m_ref, mm, i)
    _acc(s_ref, ss, i)


def _p4_body(h_ref, a2_ref, c2_ref, o_ref):
    h1 = h_ref[...].astype(jnp.float32)
    h2 = _lrelu(jnp.dot(h1, a2_ref[...], preferred_element_type=jnp.float32)
                + c2_ref[...][:1])
    rb = h1.shape[0] // 8
    o_ref[...] = jnp.sum(h2.reshape(rb, 8, 128), axis=1)


def _big_spec(nrows):
    return pl.BlockSpec((nrows, 128), lambda i: (i, 0))


def _pt_spec(width):
    return pl.BlockSpec((_RB, width), lambda i: (i, 0))


def _const_spec(shape):
    return pl.BlockSpec(shape, lambda i: (0,) * len(shape))


def _tc_call(body, n_blocks, in_arrays, in_specs, out_shape, out_specs):
    return pl.pallas_call(
        body,
        grid=(n_blocks,),
        in_specs=in_specs,
        out_specs=out_specs,
        out_shape=out_shape,
        compiler_params=pltpu.CompilerParams(
            dimension_semantics=("arbitrary",)),
    )(*in_arrays)


def _fold(w, gamma, beta, mean_in, sec_in):
    mu = w @ mean_in
    e2 = jnp.einsum('oi,ij,oj->o', w, sec_in, w)
    var = e2 - mu * mu
    scale = gamma / jnp.sqrt(var + _EPS)
    a = scale[:, None] * w
    c = beta - scale * mu
    return a, c


def _bd4(a):
    return jnp.kron(jnp.eye(4, dtype=a.dtype), a.T)


def _diag_blocks_sum(m):
    return (m[0:32, 0:32] + m[32:64, 32:64] + m[64:96, 64:96]
            + m[96:128, 96:128])


def kernel(x, xyz, idx, W_xyz, g_xyz, b_xyz, W1, g1, b1, W2, g2, b2):
    b, c, n = x.shape
    k = idx.shape[-1]
    bn = b * n
    r = bn * k
    n_blocks = bn // _RB
    rows128 = _RB * 8

    x_t = jnp.transpose(x, (0, 2, 1)).reshape(bn, c)
    xyz_t = jnp.transpose(xyz, (0, 2, 1)).reshape(bn, 3)
    table_q = jnp.pad(xyz_t, ((0, 0), (0, 29)))
    ca = jnp.concatenate([
        jnp.zeros((bn, 3), jnp.float32), xyz_t,
        jnp.ones((bn, 1), jnp.float32),
        jnp.zeros((bn, 25), jnp.float32)], axis=1)
    idx_base = jnp.arange(b, dtype=idx.dtype)[:, None, None] * n
    idx2 = (idx + idx_base).reshape(r // 128, 128).astype(jnp.int32)

    qg = _sc_gather(table_q, idx2)
    xg = _sc_gather(x_t.astype(jnp.bfloat16), idx2)
    xg128 = xg.reshape(r // 4, 128)
    qg128 = qg.reshape(r // 4, 128)

    m1, s1 = _tc_call(
        _p1_body, n_blocks,
        [qg128, ca],
        [_big_spec(rows128), _pt_spec(32)],
        [jax.ShapeDtypeStruct((128, 128), jnp.float32),
         jax.ShapeDtypeStruct((8, 128), jnp.float32)],
        [_const_spec((128, 128)), _const_spec((8, 128))],
    )
    m7 = _diag_blocks_sum(m1) / r
    egg, egc, ecc = m7[0:3, 0:3], m7[0:3, 3:6], m7[3:6, 3:6]
    mg, mc = m7[0:3, 6], m7[3:6, 6]
    ecg = egc.T
    mu9 = jnp.concatenate([mc, mg, mg - mc])
    row_c = jnp.concatenate([ecc, ecg, ecg - ecc], axis=1)
    row_g = jnp.concatenate([egc, egg, egg - egc], axis=1)
    row_d = row_g - row_c
    m9 = jnp.concatenate([row_c, row_g, row_d], axis=0)
    a_xyz, c_xyz = _fold(W_xyz, g_xyz, b_xyz, mu9, m9)
    p_mat = a_xyz[:, 0:3] - a_xyz[:, 6:9]
    q_mat = a_xyz[:, 3:6] + a_xyz[:, 6:9]
    qpad = jnp.zeros((32, 32), jnp.float32).at[:, 0:3].set(q_mat)
    qbdt = _bd4(qpad)
    uptt = (jnp.zeros((32, 32), jnp.float32)
            .at[:, 3:6].set(p_mat).at[:, 6].set(c_xyz)).T

    m2, s2, f16 = _tc_call(
        _p2_body, n_blocks,
        [xg128, qg128, x_t, ca, qbdt, uptt],
        [_big_spec(rows128), _big_spec(rows128), _pt_spec(32), _pt_spec(32),
         _const_spec((128, 128)), _const_spec((32, 32))],
        [jax.ShapeDtypeStruct((128, 128), jnp.float32),
         jax.ShapeDtypeStruct((8, 128), jnp.float32),
         jax.ShapeDtypeStruct((r // 4, 128), jnp.bfloat16)],
        [_const_spec((128, 128)), _const_spec((8, 128)),
         _big_spec(rows128)],
    )
    sec_f = _diag_blocks_sum(m2) / r
    mean_f = jnp.sum(s2.reshape(8, 4, 32), axis=(0, 1)) / r
    a1, c1 = _fold(W1, g1, b1, mean_f, sec_f)
    a1bdt = _bd4(a1)
    c1row = jnp.broadcast_to(jnp.tile(c1, 4)[None, :], (8, 128))

    m3, s3, h16 = _tc_call(
        _p3_body, n_blocks,
        [f16, a1bdt, c1row],
        [_big_spec(rows128),
         _const_spec((128, 128)), _const_spec((8, 128))],
        [jax.ShapeDtypeStruct((128, 128), jnp.float32),
         jax.ShapeDtypeStruct((8, 128), jnp.float32),
         jax.ShapeDtypeStruct((r // 4, 128), jnp.bfloat16)],
        [_const_spec((128, 128)), _const_spec((8, 128)),
         _big_spec(rows128)],
    )
    sec_h = _diag_blocks_sum(m3) / r
    mean_h = jnp.sum(s3.reshape(8, 4, 32), axis=(0, 1)) / r
    a2, c2 = _fold(W2, g2, b2, mean_h, sec_h)
    a2bdt = _bd4(a2)
    c2row = jnp.broadcast_to(jnp.tile(c2, 4)[None, :], (8, 128))

    osum = _tc_call(
        _p4_body, n_blocks,
        [h16, a2bdt, c2row],
        [_big_spec(rows128),
         _const_spec((128, 128)), _const_spec((8, 128))],
        jax.ShapeDtypeStruct((bn, 128), jnp.float32),
        _pt_spec(128),
    )
    out = jnp.sum(osum.reshape(b, n, 4, 32), axis=2) / k
    return jnp.transpose(out, (0, 2, 1))


# speedup vs baseline: 1.2712x; 1.2712x over previous
import functools

import jax
import jax.numpy as jnp
from jax import lax
from jax.experimental import pallas as pl
from jax.experimental.pallas import tpu as pltpu
from jax.experimental.pallas import tpu_sc as plsc

_SLOPE = 0.2
_EPS = 1e-5


_NW = 32
_SUB = 128
_NSUB = 8


def _sc_gather(table, idx2):
    dt = table.dtype
    n_rows_idx, lw = idx2.shape
    r_total = n_rows_idx * lw
    ch = _SUB * _NSUB
    per_w = r_total // _NW
    n_it = per_w // ch

    mesh = plsc.VectorSubcoreMesh(core_axis_name="c", subcore_axis_name="s")

    @functools.partial(
        pl.kernel,
        mesh=mesh,
        out_type=jax.ShapeDtypeStruct((r_total, 32), dt),
        scratch_types=[
            pltpu.VMEM((_NSUB, _SUB), jnp.int32),
            pltpu.VMEM((ch, 32), dt),
            pltpu.SemaphoreType.DMA,
        ],
        compiler_params=pltpu.CompilerParams(use_tc_tiling_on_sc=False),
    )
    def gather_kernel(t_hbm, idx_hbm, o_hbm, idx_v, r_v, sem):
        wid = lax.axis_index("s") * 2 + lax.axis_index("c")
        base_w = wid * per_w

        def body(it, carry):
            base = pl.multiple_of(base_w + it * ch, ch)
            idx_row = pl.multiple_of(base // lw, _NSUB)
            pltpu.sync_copy(idx_hbm.at[pl.ds(idx_row, _NSUB)], idx_v)
            copies = []
            for j in range(_NSUB):
                copies.append(pltpu.async_copy(
                    t_hbm.at[idx_v.at[j]],
                    r_v.at[pl.ds(j * _SUB, _SUB)], sem))
            for cp in copies:
                cp.wait()
            pltpu.sync_copy(r_v, o_hbm.at[pl.ds(base, ch)])
            return carry

        lax.fori_loop(0, n_it, body, 0)

    return gather_kernel(table, idx2)


_RB = 256


def _lrelu(v):
    return jnp.where(v >= 0, v, _SLOPE * v)


def _feature(xg, qg, xc, ca, qbdt, uptt):
    rb = xc.shape[0]
    u_pt = jnp.dot(ca, uptt, preferred_element_type=jnp.float32)
    cent = u_pt - xc
    cent = jnp.concatenate([cent, cent, cent, cent], axis=1)
    cent = jnp.broadcast_to(cent[:, None, :], (rb, 8, 128))
    cent = cent.reshape(rb * 8, 128)
    qpart = jnp.dot(qg, qbdt, preferred_element_type=jnp.float32)
    return _lrelu(xg.astype(jnp.float32) + qpart + cent)


def _moments(v):
    mm = lax.dot_general(v, v, (((0,), (0,)), ((), ())),
                         preferred_element_type=jnp.float32)
    m = v.shape[0]
    ss = jnp.sum(v.reshape(8, m // 8, 128), axis=1)
    return mm, ss


def _acc(ref, val, step):
    @pl.when(step == 0)
    def _():
        ref[...] = val

    @pl.when(step != 0)
    def _():
        ref[...] += val


def _p1_body(qg_ref, ca_ref, m_ref, s_ref):
    i = pl.program_id(0)
    rb = ca_ref.shape[0]
    ca = ca_ref[...]
    cat = jnp.concatenate([ca, ca, ca, ca], axis=1)
    cat = jnp.broadcast_to(cat[:, None, :], (rb, 8, 128)).reshape(rb * 8, 128)
    w = qg_ref[...] + cat
    mm, ss = _moments(w)
    _acc(m_ref, mm, i)
    _acc(s_ref, ss, i)


def _p2_body(xg_ref, qg_ref, xc_ref, ca_ref, qbdt_ref, uptt_ref,
             m_ref, s_ref, f_ref):
    i = pl.program_id(0)
    f = _feature(xg_ref[...], qg_ref[...], xc_ref[...], ca_ref[...],
                 qbdt_ref[...], uptt_ref[...])
    f_ref[...] = f.astype(jnp.bfloat16)
    mm, ss = _moments(f)
    _acc(m_ref, mm, i)
    _acc(s_ref, ss, i)


def _p3_body(f_ref, a1_ref, c1_ref, m_ref, s_ref, h_ref):
    i = pl.program_id(0)
    f = f_ref[...].astype(jnp.float32)
    h1 = _lrelu(jnp.dot(f, a1_ref[...], preferred_element_type=jnp.float32)
                + c1_ref[...][:1])
    h_ref[...] = h1.astype(jnp.bfloat16)
    mm, ss = _moments(h1)
    _acc(m_ref, mm, i)
    _acc(s_ref, ss, i)


def _p4_body(h_ref, a2_ref, c2_ref, o_ref):
    h1 = h_ref[...].astype(jnp.float32)
    h2 = _lrelu(jnp.dot(h1, a2_ref[...], preferred_element_type=jnp.float32)
                + c2_ref[...][:1])
    rb = h1.shape[0] // 8
    o_ref[...] = jnp.sum(h2.reshape(rb, 8, 128), axis=1)


def _big_spec(nrows):
    return pl.BlockSpec((nrows, 128), lambda i: (i, 0))


def _pt_spec(width):
    return pl.BlockSpec((_RB, width), lambda i: (i, 0))


def _const_spec(shape):
    return pl.BlockSpec(shape, lambda i: (0,) * len(shape))


def _tc_call(body, n_blocks, in_arrays, in_specs, out_shape, out_specs):
    return pl.pallas_call(
        body,
        grid=(n_blocks,),
        in_specs=in_specs,
        out_specs=out_specs,
        out_shape=out_shape,
        compiler_params=pltpu.CompilerParams(
            dimension_semantics=("arbitrary",)),
    )(*in_arrays)


def _fold(w, gamma, beta, mean_in, sec_in):
    mu = w @ mean_in
    e2 = jnp.einsum('oi,ij,oj->o', w, sec_in, w)
    var = e2 - mu * mu
    scale = gamma / jnp.sqrt(var + _EPS)
    a = scale[:, None] * w
    c = beta - scale * mu
    return a, c


def _bd4(a):
    return jnp.kron(jnp.eye(4, dtype=a.dtype), a.T)


def _diag_blocks_sum(m):
    return (m[0:32, 0:32] + m[32:64, 32:64] + m[64:96, 64:96]
            + m[96:128, 96:128])


def kernel(x, xyz, idx, W_xyz, g_xyz, b_xyz, W1, g1, b1, W2, g2, b2):
    b, c, n = x.shape
    k = idx.shape[-1]
    bn = b * n
    r = bn * k
    n_blocks = bn // _RB
    rows128 = _RB * 8

    x_t = jnp.transpose(x, (0, 2, 1)).reshape(bn, c)
    xyz_t = jnp.transpose(xyz, (0, 2, 1)).reshape(bn, 3)
    table_q = jnp.pad(xyz_t, ((0, 0), (0, 29)))
    ca = jnp.concatenate([
        jnp.zeros((bn, 3), jnp.float32), xyz_t,
        jnp.ones((bn, 1), jnp.float32),
        jnp.zeros((bn, 25), jnp.float32)], axis=1)
    idx_base = jnp.arange(b, dtype=idx.dtype)[:, None, None] * n
    idx2 = (idx + idx_base).reshape(r // 128, 128).astype(jnp.int32)

    qg = _sc_gather(table_q, idx2)
    xg = _sc_gather(x_t, idx2)
    xg128 = xg.reshape(r // 4, 128)
    qg128 = qg.reshape(r // 4, 128)

    m1, s1 = _tc_call(
        _p1_body, n_blocks,
        [qg128, ca],
        [_big_spec(rows128), _pt_spec(32)],
        [jax.ShapeDtypeStruct((128, 128), jnp.float32),
         jax.ShapeDtypeStruct((8, 128), jnp.float32)],
        [_const_spec((128, 128)), _const_spec((8, 128))],
    )
    m7 = _diag_blocks_sum(m1) / r
    egg, egc, ecc = m7[0:3, 0:3], m7[0:3, 3:6], m7[3:6, 3:6]
    mg, mc = m7[0:3, 6], m7[3:6, 6]
    ecg = egc.T
    mu9 = jnp.concatenate([mc, mg, mg - mc])
    row_c = jnp.concatenate([ecc, ecg, ecg - ecc], axis=1)
    row_g = jnp.concatenate([egc, egg, egg - egc], axis=1)
    row_d = row_g - row_c
    m9 = jnp.concatenate([row_c, row_g, row_d], axis=0)
    a_xyz, c_xyz = _fold(W_xyz, g_xyz, b_xyz, mu9, m9)
    p_mat = a_xyz[:, 0:3] - a_xyz[:, 6:9]
    q_mat = a_xyz[:, 3:6] + a_xyz[:, 6:9]
    qpad = jnp.zeros((32, 32), jnp.float32).at[:, 0:3].set(q_mat)
    qbdt = _bd4(qpad)
    uptt = (jnp.zeros((32, 32), jnp.float32)
            .at[:, 3:6].set(p_mat).at[:, 6].set(c_xyz)).T

    m2, s2, f16 = _tc_call(
        _p2_body, n_blocks,
        [xg128, qg128, x_t, ca, qbdt, uptt],
        [_big_spec(rows128), _big_spec(rows128), _pt_spec(32), _pt_spec(32),
         _const_spec((128, 128)), _const_spec((32, 32))],
        [jax.ShapeDtypeStruct((128, 128), jnp.float32),
         jax.ShapeDtypeStruct((8, 128), jnp.float32),
         jax.ShapeDtypeStruct((r // 4, 128), jnp.bfloat16)],
        [_const_spec((128, 128)), _const_spec((8, 128)),
         _big_spec(rows128)],
    )
    sec_f = _diag_blocks_sum(m2) / r
    mean_f = jnp.sum(s2.reshape(8, 4, 32), axis=(0, 1)) / r
    a1, c1 = _fold(W1, g1, b1, mean_f, sec_f)
    a1bdt = _bd4(a1)
    c1row = jnp.broadcast_to(jnp.tile(c1, 4)[None, :], (8, 128))

    m3, s3, h16 = _tc_call(
        _p3_body, n_blocks,
        [f16, a1bdt, c1row],
        [_big_spec(rows128),
         _const_spec((128, 128)), _const_spec((8, 128))],
        [jax.ShapeDtypeStruct((128, 128), jnp.float32),
         jax.ShapeDtypeStruct((8, 128), jnp.float32),
         jax.ShapeDtypeStruct((r // 4, 128), jnp.bfloat16)],
        [_const_spec((128, 128)), _const_spec((8, 128)),
         _big_spec(rows128)],
    )
    sec_h = _diag_blocks_sum(m3) / r
    mean_h = jnp.sum(s3.reshape(8, 4, 32), axis=(0, 1)) / r
    a2, c2 = _fold(W2, g2, b2, mean_h, sec_h)
    a2bdt = _bd4(a2)
    c2row = jnp.broadcast_to(jnp.tile(c2, 4)[None, :], (8, 128))

    osum = _tc_call(
        _p4_body, n_blocks,
        [h16, a2bdt, c2row],
        [_big_spec(rows128),
         _const_spec((128, 128)), _const_spec((8, 128))],
        jax.ShapeDtypeStruct((bn, 128), jnp.float32),
        _pt_spec(128),
    )
    out = jnp.sum(osum.reshape(b, n, 4, 32), axis=2) / k
    return jnp.transpose(out, (0, 2, 1))


# speedup vs baseline: 1.4681x; 1.1549x over previous
import functools

import jax
import jax.numpy as jnp
from jax import lax
from jax.experimental import pallas as pl
from jax.experimental.pallas import tpu as pltpu
from jax.experimental.pallas import tpu_sc as plsc

_SLOPE = 0.2
_EPS = 1e-5


_NW = 32
_SUB = 128
_NSUB = 8


def _sc_gather(table, idx2):
    dt = table.dtype
    n_rows_idx, lw = idx2.shape
    r_total = n_rows_idx * lw
    ch = _SUB * _NSUB
    per_w = r_total // _NW
    n_it = per_w // ch

    mesh = plsc.VectorSubcoreMesh(core_axis_name="c", subcore_axis_name="s")

    @functools.partial(
        pl.kernel,
        mesh=mesh,
        out_type=jax.ShapeDtypeStruct((r_total, 32), dt),
        scratch_types=[
            pltpu.VMEM((_NSUB, _SUB), jnp.int32),
            pltpu.VMEM((ch, 32), dt),
            pltpu.SemaphoreType.DMA,
        ],
        compiler_params=pltpu.CompilerParams(use_tc_tiling_on_sc=False),
    )
    def gather_kernel(t_hbm, idx_hbm, o_hbm, idx_v, r_v, sem):
        wid = lax.axis_index("s") * 2 + lax.axis_index("c")
        base_w = wid * per_w

        def body(it, carry):
            base = pl.multiple_of(base_w + it * ch, ch)
            idx_row = pl.multiple_of(base // lw, _NSUB)
            pltpu.sync_copy(idx_hbm.at[pl.ds(idx_row, _NSUB)], idx_v)
            copies = []
            for j in range(_NSUB):
                copies.append(pltpu.async_copy(
                    t_hbm.at[idx_v.at[j]],
                    r_v.at[pl.ds(j * _SUB, _SUB)], sem))
            for cp in copies:
                cp.wait()
            pltpu.sync_copy(r_v, o_hbm.at[pl.ds(base, ch)])
            return carry

        lax.fori_loop(0, n_it, body, 0)

    return gather_kernel(table, idx2)


_RB = 512


def _lrelu(v):
    return jnp.where(v >= 0, v, _SLOPE * v)


def _feature(xg, qg, xc, ca, qbdt, uptt):
    rb = xc.shape[0]
    u_pt = jnp.dot(ca, uptt, preferred_element_type=jnp.float32)
    cent = u_pt - xc
    cent = jnp.concatenate([cent, cent, cent, cent], axis=1)
    cent = jnp.broadcast_to(cent[:, None, :], (rb, 8, 128))
    cent = cent.reshape(rb * 8, 128)
    qpart = jnp.dot(qg, qbdt, preferred_element_type=jnp.float32)
    return _lrelu(xg.astype(jnp.float32) + qpart + cent)


def _moments(v):
    mm = lax.dot_general(v, v, (((0,), (0,)), ((), ())),
                         preferred_element_type=jnp.float32)
    m = v.shape[0]
    ss = jnp.sum(v.reshape(8, m // 8, 128), axis=1)
    return mm, ss


def _acc(ref, val, step):
    @pl.when(step == 0)
    def _():
        ref[...] = val

    @pl.when(step != 0)
    def _():
        ref[...] += val


def _p1_body(qg_ref, ca_ref, m_ref, s_ref):
    i = pl.program_id(0)
    rb = ca_ref.shape[0]
    ca = ca_ref[...]
    cat = jnp.concatenate([ca, ca, ca, ca], axis=1)
    cat = jnp.broadcast_to(cat[:, None, :], (rb, 8, 128)).reshape(rb * 8, 128)
    w = qg_ref[...] + cat
    mm, ss = _moments(w)
    _acc(m_ref, mm, i)
    _acc(s_ref, ss, i)


def _p2_body(xg_ref, qg_ref, xc_ref, ca_ref, qbdt_ref, uptt_ref,
             m_ref, s_ref, f_ref):
    i = pl.program_id(0)
    f = _feature(xg_ref[...], qg_ref[...], xc_ref[...], ca_ref[...],
                 qbdt_ref[...], uptt_ref[...])
    f_ref[...] = f.astype(jnp.bfloat16)
    mm, ss = _moments(f)
    _acc(m_ref, mm, i)
    _acc(s_ref, ss, i)


def _p3_body(f_ref, a1_ref, c1_ref, m_ref, s_ref, h_ref):
    i = pl.program_id(0)
    f = f_ref[...].astype(jnp.float32)
    h1 = _lrelu(jnp.dot(f, a1_ref[...], preferred_element_type=jnp.float32)
                + c1_ref[...][:1])
    h_ref[...] = h1.astype(jnp.bfloat16)
    mm, ss = _moments(h1)
    _acc(m_ref, mm, i)
    _acc(s_ref, ss, i)


def _p4_body(h_ref, a2_ref, c2_ref, o_ref):
    h1 = h_ref[...].astype(jnp.float32)
    h2 = _lrelu(jnp.dot(h1, a2_ref[...], preferred_element_type=jnp.float32)
                + c2_ref[...][:1])
    rb = h1.shape[0] // 8
    o_ref[...] = jnp.sum(h2.reshape(rb, 8, 128), axis=1)


def _big_spec(nrows):
    return pl.BlockSpec((nrows, 128), lambda i: (i, 0))


def _pt_spec(width):
    return pl.BlockSpec((_RB, width), lambda i: (i, 0))


def _const_spec(shape):
    return pl.BlockSpec(shape, lambda i: (0,) * len(shape))


def _tc_call(body, n_blocks, in_arrays, in_specs, out_shape, out_specs):
    return pl.pallas_call(
        body,
        grid=(n_blocks,),
        in_specs=in_specs,
        out_specs=out_specs,
        out_shape=out_shape,
        compiler_params=pltpu.CompilerParams(
            dimension_semantics=("arbitrary",)),
    )(*in_arrays)


def _fold(w, gamma, beta, mean_in, sec_in):
    mu = w @ mean_in
    e2 = jnp.einsum('oi,ij,oj->o', w, sec_in, w)
    var = e2 - mu * mu
    scale = gamma / jnp.sqrt(var + _EPS)
    a = scale[:, None] * w
    c = beta - scale * mu
    return a, c


def _bd4(a):
    return jnp.kron(jnp.eye(4, dtype=a.dtype), a.T)


def _diag_blocks_sum(m):
    return (m[0:32, 0:32] + m[32:64, 32:64] + m[64:96, 64:96]
            + m[96:128, 96:128])


def kernel(x, xyz, idx, W_xyz, g_xyz, b_xyz, W1, g1, b1, W2, g2, b2):
    b, c, n = x.shape
    k = idx.shape[-1]
    bn = b * n
    r = bn * k
    n_blocks = bn // _RB
    rows128 = _RB * 8

    x_t = jnp.transpose(x, (0, 2, 1)).reshape(bn, c)
    xyz_t = jnp.transpose(xyz, (0, 2, 1)).reshape(bn, 3)
    table_q = jnp.pad(xyz_t, ((0, 0), (0, 29)))
    ca = jnp.concatenate([
        jnp.zeros((bn, 3), jnp.float32), xyz_t,
        jnp.ones((bn, 1), jnp.float32),
        jnp.zeros((bn, 25), jnp.float32)], axis=1)
    idx_base = jnp.arange(b, dtype=idx.dtype)[:, None, None] * n
    idx2 = (idx + idx_base).reshape(r // 128, 128).astype(jnp.int32)

    qg = _sc_gather(table_q, idx2)
    xg = _sc_gather(x_t, idx2)
    xg128 = xg.reshape(r // 4, 128)
    qg128 = qg.reshape(r // 4, 128)

    m1, s1 = _tc_call(
        _p1_body, n_blocks,
        [qg128, ca],
        [_big_spec(rows128), _pt_spec(32)],
        [jax.ShapeDtypeStruct((128, 128), jnp.float32),
         jax.ShapeDtypeStruct((8, 128), jnp.float32)],
        [_const_spec((128, 128)), _const_spec((8, 128))],
    )
    m7 = _diag_blocks_sum(m1) / r
    egg, egc, ecc = m7[0:3, 0:3], m7[0:3, 3:6], m7[3:6, 3:6]
    mg, mc = m7[0:3, 6], m7[3:6, 6]
    ecg = egc.T
    mu9 = jnp.concatenate([mc, mg, mg - mc])
    row_c = jnp.concatenate([ecc, ecg, ecg - ecc], axis=1)
    row_g = jnp.concatenate([egc, egg, egg - egc], axis=1)
    row_d = row_g - row_c
    m9 = jnp.concatenate([row_c, row_g, row_d], axis=0)
    a_xyz, c_xyz = _fold(W_xyz, g_xyz, b_xyz, mu9, m9)
    p_mat = a_xyz[:, 0:3] - a_xyz[:, 6:9]
    q_mat = a_xyz[:, 3:6] + a_xyz[:, 6:9]
    qpad = jnp.zeros((32, 32), jnp.float32).at[:, 0:3].set(q_mat)
    qbdt = _bd4(qpad)
    uptt = (jnp.zeros((32, 32), jnp.float32)
            .at[:, 3:6].set(p_mat).at[:, 6].set(c_xyz)).T

    m2, s2, f16 = _tc_call(
        _p2_body, n_blocks,
        [xg128, qg128, x_t, ca, qbdt, uptt],
        [_big_spec(rows128), _big_spec(rows128), _pt_spec(32), _pt_spec(32),
         _const_spec((128, 128)), _const_spec((32, 32))],
        [jax.ShapeDtypeStruct((128, 128), jnp.float32),
         jax.ShapeDtypeStruct((8, 128), jnp.float32),
         jax.ShapeDtypeStruct((r // 4, 128), jnp.bfloat16)],
        [_const_spec((128, 128)), _const_spec((8, 128)),
         _big_spec(rows128)],
    )
    sec_f = _diag_blocks_sum(m2) / r
    mean_f = jnp.sum(s2.reshape(8, 4, 32), axis=(0, 1)) / r
    a1, c1 = _fold(W1, g1, b1, mean_f, sec_f)
    a1bdt = _bd4(a1)
    c1row = jnp.broadcast_to(jnp.tile(c1, 4)[None, :], (8, 128))

    m3, s3, h16 = _tc_call(
        _p3_body, n_blocks,
        [f16, a1bdt, c1row],
        [_big_spec(rows128),
         _const_spec((128, 128)), _const_spec((8, 128))],
        [jax.ShapeDtypeStruct((128, 128), jnp.float32),
         jax.ShapeDtypeStruct((8, 128), jnp.float32),
         jax.ShapeDtypeStruct((r // 4, 128), jnp.bfloat16)],
        [_const_spec((128, 128)), _const_spec((8, 128)),
         _big_spec(rows128)],
    )
    sec_h = _diag_blocks_sum(m3) / r
    mean_h = jnp.sum(s3.reshape(8, 4, 32), axis=(0, 1)) / r
    a2, c2 = _fold(W2, g2, b2, mean_h, sec_h)
    a2bdt = _bd4(a2)
    c2row = jnp.broadcast_to(jnp.tile(c2, 4)[None, :], (8, 128))

    osum = _tc_call(
        _p4_body, n_blocks,
        [h16, a2bdt, c2row],
        [_big_spec(rows128),
         _const_spec((128, 128)), _const_spec((8, 128))],
        jax.ShapeDtypeStruct((bn, 128), jnp.float32),
        _pt_spec(128),
    )
    out = jnp.sum(osum.reshape(b, n, 4, 32), axis=2) / k
    return jnp.transpose(out, (0, 2, 1))


# speedup vs baseline: 1.5902x; 1.0832x over previous
import functools

import jax
import jax.numpy as jnp
from jax import lax
from jax.experimental import pallas as pl
from jax.experimental.pallas import tpu as pltpu
from jax.experimental.pallas import tpu_sc as plsc

_SLOPE = 0.2
_EPS = 1e-5


_NW = 32
_SUB = 128
_NSUB = 8


def _sc_gather(table, idx2):
    dt = table.dtype
    n_rows_idx, lw = idx2.shape
    r_total = n_rows_idx * lw
    ch = _SUB * _NSUB
    per_w = r_total // _NW
    n_it = per_w // ch

    mesh = plsc.VectorSubcoreMesh(core_axis_name="c", subcore_axis_name="s")

    @functools.partial(
        pl.kernel,
        mesh=mesh,
        out_type=jax.ShapeDtypeStruct((r_total, 32), dt),
        scratch_types=[
            pltpu.VMEM((_NSUB, _SUB), jnp.int32),
            pltpu.VMEM((ch, 32), dt),
            pltpu.SemaphoreType.DMA,
        ],
        compiler_params=pltpu.CompilerParams(use_tc_tiling_on_sc=False),
    )
    def gather_kernel(t_hbm, idx_hbm, o_hbm, idx_v, r_v, sem):
        wid = lax.axis_index("s") * 2 + lax.axis_index("c")
        base_w = wid * per_w

        def body(it, carry):
            base = pl.multiple_of(base_w + it * ch, ch)
            idx_row = pl.multiple_of(base // lw, _NSUB)
            pltpu.sync_copy(idx_hbm.at[pl.ds(idx_row, _NSUB)], idx_v)
            copies = []
            for j in range(_NSUB):
                copies.append(pltpu.async_copy(
                    t_hbm.at[idx_v.at[j]],
                    r_v.at[pl.ds(j * _SUB, _SUB)], sem))
            for cp in copies:
                cp.wait()
            pltpu.sync_copy(r_v, o_hbm.at[pl.ds(base, ch)])
            return carry

        lax.fori_loop(0, n_it, body, 0)

    return gather_kernel(table, idx2)


_RB = 1024


def _lrelu(v):
    return jnp.where(v >= 0, v, _SLOPE * v)


def _feature(xg, qg, xc, ca, qbdt, uptt):
    rb = xc.shape[0]
    u_pt = jnp.dot(ca, uptt, preferred_element_type=jnp.float32)
    cent = u_pt - xc
    cent = jnp.concatenate([cent, cent, cent, cent], axis=1)
    cent = jnp.broadcast_to(cent[:, None, :], (rb, 8, 128))
    cent = cent.reshape(rb * 8, 128)
    qpart = jnp.dot(qg, qbdt, preferred_element_type=jnp.float32)
    return _lrelu(xg.astype(jnp.float32) + qpart + cent)


def _moments(v):
    mm = lax.dot_general(v, v, (((0,), (0,)), ((), ())),
                         preferred_element_type=jnp.float32)
    m = v.shape[0]
    ss = jnp.sum(v.reshape(8, m // 8, 128), axis=1)
    return mm, ss


def _acc(ref, val, step):
    @pl.when(step == 0)
    def _():
        ref[...] = val

    @pl.when(step != 0)
    def _():
        ref[...] += val


def _p1_body(qg_ref, ca_ref, m_ref, s_ref):
    i = pl.program_id(0)
    rb = ca_ref.shape[0]
    ca = ca_ref[...]
    cat = jnp.concatenate([ca, ca, ca, ca], axis=1)
    cat = jnp.broadcast_to(cat[:, None, :], (rb, 8, 128)).reshape(rb * 8, 128)
    w = qg_ref[...] + cat
    mm, ss = _moments(w)
    _acc(m_ref, mm, i)
    _acc(s_ref, ss, i)


def _p2_body(xg_ref, qg_ref, xc_ref, ca_ref, qbdt_ref, uptt_ref,
             m_ref, s_ref, f_ref):
    i = pl.program_id(0)
    f = _feature(xg_ref[...], qg_ref[...], xc_ref[...], ca_ref[...],
                 qbdt_ref[...], uptt_ref[...])
    f_ref[...] = f.astype(jnp.bfloat16)
    mm, ss = _moments(f)
    _acc(m_ref, mm, i)
    _acc(s_ref, ss, i)


def _p3_body(f_ref, a1_ref, c1_ref, m_ref, s_ref, h_ref):
    i = pl.program_id(0)
    f = f_ref[...].astype(jnp.float32)
    h1 = _lrelu(jnp.dot(f, a1_ref[...], preferred_element_type=jnp.float32)
                + c1_ref[...][:1])
    h_ref[...] = h1.astype(jnp.bfloat16)
    mm, ss = _moments(h1)
    _acc(m_ref, mm, i)
    _acc(s_ref, ss, i)


def _p4_body(h_ref, a2_ref, c2_ref, o_ref):
    h1 = h_ref[...].astype(jnp.float32)
    h2 = _lrelu(jnp.dot(h1, a2_ref[...], preferred_element_type=jnp.float32)
                + c2_ref[...][:1])
    rb = h1.shape[0] // 8
    o_ref[...] = jnp.sum(h2.reshape(rb, 8, 128), axis=1)


def _big_spec(nrows):
    return pl.BlockSpec((nrows, 128), lambda i: (i, 0))


def _pt_spec(width):
    return pl.BlockSpec((_RB, width), lambda i: (i, 0))


def _const_spec(shape):
    return pl.BlockSpec(shape, lambda i: (0,) * len(shape))


def _tc_call(body, n_blocks, in_arrays, in_specs, out_shape, out_specs):
    return pl.pallas_call(
        body,
        grid=(n_blocks,),
        in_specs=in_specs,
        out_specs=out_specs,
        out_shape=out_shape,
        compiler_params=pltpu.CompilerParams(
            dimension_semantics=("arbitrary",)),
    )(*in_arrays)


def _fold(w, gamma, beta, mean_in, sec_in):
    mu = w @ mean_in
    e2 = jnp.einsum('oi,ij,oj->o', w, sec_in, w)
    var = e2 - mu * mu
    scale = gamma / jnp.sqrt(var + _EPS)
    a = scale[:, None] * w
    c = beta - scale * mu
    return a, c


def _bd4(a):
    return jnp.kron(jnp.eye(4, dtype=a.dtype), a.T)


def _diag_blocks_sum(m):
    return (m[0:32, 0:32] + m[32:64, 32:64] + m[64:96, 64:96]
            + m[96:128, 96:128])


def kernel(x, xyz, idx, W_xyz, g_xyz, b_xyz, W1, g1, b1, W2, g2, b2):
    b, c, n = x.shape
    k = idx.shape[-1]
    bn = b * n
    r = bn * k
    n_blocks = bn // _RB
    rows128 = _RB * 8

    x_t = jnp.transpose(x, (0, 2, 1)).reshape(bn, c)
    xyz_t = jnp.transpose(xyz, (0, 2, 1)).reshape(bn, 3)
    table_q = jnp.pad(xyz_t, ((0, 0), (0, 29)))
    ca = jnp.concatenate([
        jnp.zeros((bn, 3), jnp.float32), xyz_t,
        jnp.ones((bn, 1), jnp.float32),
        jnp.zeros((bn, 25), jnp.float32)], axis=1)
    idx_base = jnp.arange(b, dtype=idx.dtype)[:, None, None] * n
    idx2 = (idx + idx_base).reshape(r // 128, 128).astype(jnp.int32)

    qg = _sc_gather(table_q, idx2)
    xg = _sc_gather(x_t, idx2)
    xg128 = xg.reshape(r // 4, 128)
    qg128 = qg.reshape(r // 4, 128)

    m1, s1 = _tc_call(
        _p1_body, n_blocks,
        [qg128, ca],
        [_big_spec(rows128), _pt_spec(32)],
        [jax.ShapeDtypeStruct((128, 128), jnp.float32),
         jax.ShapeDtypeStruct((8, 128), jnp.float32)],
        [_const_spec((128, 128)), _const_spec((8, 128))],
    )
    m7 = _diag_blocks_sum(m1) / r
    egg, egc, ecc = m7[0:3, 0:3], m7[0:3, 3:6], m7[3:6, 3:6]
    mg, mc = m7[0:3, 6], m7[3:6, 6]
    ecg = egc.T
    mu9 = jnp.concatenate([mc, mg, mg - mc])
    row_c = jnp.concatenate([ecc, ecg, ecg - ecc], axis=1)
    row_g = jnp.concatenate([egc, egg, egg - egc], axis=1)
    row_d = row_g - row_c
    m9 = jnp.concatenate([row_c, row_g, row_d], axis=0)
    a_xyz, c_xyz = _fold(W_xyz, g_xyz, b_xyz, mu9, m9)
    p_mat = a_xyz[:, 0:3] - a_xyz[:, 6:9]
    q_mat = a_xyz[:, 3:6] + a_xyz[:, 6:9]
    qpad = jnp.zeros((32, 32), jnp.float32).at[:, 0:3].set(q_mat)
    qbdt = _bd4(qpad)
    uptt = (jnp.zeros((32, 32), jnp.float32)
            .at[:, 3:6].set(p_mat).at[:, 6].set(c_xyz)).T

    m2, s2, f16 = _tc_call(
        _p2_body, n_blocks,
        [xg128, qg128, x_t, ca, qbdt, uptt],
        [_big_spec(rows128), _big_spec(rows128), _pt_spec(32), _pt_spec(32),
         _const_spec((128, 128)), _const_spec((32, 32))],
        [jax.ShapeDtypeStruct((128, 128), jnp.float32),
         jax.ShapeDtypeStruct((8, 128), jnp.float32),
         jax.ShapeDtypeStruct((r // 4, 128), jnp.bfloat16)],
        [_const_spec((128, 128)), _const_spec((8, 128)),
         _big_spec(rows128)],
    )
    sec_f = _diag_blocks_sum(m2) / r
    mean_f = jnp.sum(s2.reshape(8, 4, 32), axis=(0, 1)) / r
    a1, c1 = _fold(W1, g1, b1, mean_f, sec_f)
    a1bdt = _bd4(a1)
    c1row = jnp.broadcast_to(jnp.tile(c1, 4)[None, :], (8, 128))

    m3, s3, h16 = _tc_call(
        _p3_body, n_blocks,
        [f16, a1bdt, c1row],
        [_big_spec(rows128),
         _const_spec((128, 128)), _const_spec((8, 128))],
        [jax.ShapeDtypeStruct((128, 128), jnp.float32),
         jax.ShapeDtypeStruct((8, 128), jnp.float32),
         jax.ShapeDtypeStruct((r // 4, 128), jnp.bfloat16)],
        [_const_spec((128, 128)), _const_spec((8, 128)),
         _big_spec(rows128)],
    )
    sec_h = _diag_blocks_sum(m3) / r
    mean_h = jnp.sum(s3.reshape(8, 4, 32), axis=(0, 1)) / r
    a2, c2 = _fold(W2, g2, b2, mean_h, sec_h)
    a2bdt = _bd4(a2)
    c2row = jnp.broadcast_to(jnp.tile(c2, 4)[None, :], (8, 128))

    osum = _tc_call(
        _p4_body, n_blocks,
        [h16, a2bdt, c2row],
        [_big_spec(rows128),
         _const_spec((128, 128)), _const_spec((8, 128))],
        jax.ShapeDtypeStruct((bn, 128), jnp.float32),
        _pt_spec(128),
    )
    out = jnp.sum(osum.reshape(b, n, 4, 32), axis=2) / k
    return jnp.transpose(out, (0, 2, 1))


# speedup vs baseline: 1.6201x; 1.0188x over previous
import functools

import jax
import jax.numpy as jnp
from jax import lax
from jax.experimental import pallas as pl
from jax.experimental.pallas import tpu as pltpu
from jax.experimental.pallas import tpu_sc as plsc

_SLOPE = 0.2
_EPS = 1e-5


_NW = 32
_SUB = 128
_NSUB = 8


def _sc_gather(table, idx2):
    dt = table.dtype
    n_rows_idx, lw = idx2.shape
    r_total = n_rows_idx * lw
    ch = _SUB * _NSUB
    per_w = r_total // _NW
    n_it = per_w // ch

    mesh = plsc.VectorSubcoreMesh(core_axis_name="c", subcore_axis_name="s")

    @functools.partial(
        pl.kernel,
        mesh=mesh,
        out_type=jax.ShapeDtypeStruct((r_total, 32), dt),
        scratch_types=[
            pltpu.VMEM((_NSUB, _SUB), jnp.int32),
            pltpu.VMEM((ch, 32), dt),
            pltpu.SemaphoreType.DMA,
        ],
        compiler_params=pltpu.CompilerParams(use_tc_tiling_on_sc=False),
    )
    def gather_kernel(t_hbm, idx_hbm, o_hbm, idx_v, r_v, sem):
        wid = lax.axis_index("s") * 2 + lax.axis_index("c")
        base_w = wid * per_w

        def body(it, carry):
            base = pl.multiple_of(base_w + it * ch, ch)
            idx_row = pl.multiple_of(base // lw, _NSUB)
            pltpu.sync_copy(idx_hbm.at[pl.ds(idx_row, _NSUB)], idx_v)
            copies = []
            for j in range(_NSUB):
                copies.append(pltpu.async_copy(
                    t_hbm.at[idx_v.at[j]],
                    r_v.at[pl.ds(j * _SUB, _SUB)], sem))
            for cp in copies:
                cp.wait()
            pltpu.sync_copy(r_v, o_hbm.at[pl.ds(base, ch)])
            return carry

        lax.fori_loop(0, n_it, body, 0)

    return gather_kernel(table, idx2)


_RB = 2048


def _lrelu(v):
    return jnp.where(v >= 0, v, _SLOPE * v)


def _feature(xg, qg, xc, ca, qbdt, uptt):
    rb = xc.shape[0]
    u_pt = jnp.dot(ca, uptt, preferred_element_type=jnp.float32)
    cent = u_pt - xc
    cent = jnp.concatenate([cent, cent, cent, cent], axis=1)
    cent = jnp.broadcast_to(cent[:, None, :], (rb, 8, 128))
    cent = cent.reshape(rb * 8, 128)
    qpart = jnp.dot(qg, qbdt, preferred_element_type=jnp.float32)
    return _lrelu(xg.astype(jnp.float32) + qpart + cent)


def _moments(v):
    mm = lax.dot_general(v, v, (((0,), (0,)), ((), ())),
                         preferred_element_type=jnp.float32)
    m = v.shape[0]
    ss = jnp.sum(v.reshape(8, m // 8, 128), axis=1)
    return mm, ss


def _acc(ref, val, step):
    @pl.when(step == 0)
    def _():
        ref[...] = val

    @pl.when(step != 0)
    def _():
        ref[...] += val


def _p1_body(qg_ref, ca_ref, m_ref, s_ref):
    i = pl.program_id(0)
    rb = ca_ref.shape[0]
    ca = ca_ref[...]
    cat = jnp.concatenate([ca, ca, ca, ca], axis=1)
    cat = jnp.broadcast_to(cat[:, None, :], (rb, 8, 128)).reshape(rb * 8, 128)
    w = qg_ref[...] + cat
    mm, ss = _moments(w)
    _acc(m_ref, mm, i)
    _acc(s_ref, ss, i)


def _p2_body(xg_ref, qg_ref, xc_ref, ca_ref, qbdt_ref, uptt_ref,
             m_ref, s_ref, f_ref):
    i = pl.program_id(0)
    f = _feature(xg_ref[...], qg_ref[...], xc_ref[...], ca_ref[...],
                 qbdt_ref[...], uptt_ref[...])
    f_ref[...] = f.astype(jnp.bfloat16)
    mm, ss = _moments(f)
    _acc(m_ref, mm, i)
    _acc(s_ref, ss, i)


def _p3_body(f_ref, a1_ref, c1_ref, m_ref, s_ref, h_ref):
    i = pl.program_id(0)
    f = f_ref[...].astype(jnp.float32)
    h1 = _lrelu(jnp.dot(f, a1_ref[...], preferred_element_type=jnp.float32)
                + c1_ref[...][:1])
    h_ref[...] = h1.astype(jnp.bfloat16)
    mm, ss = _moments(h1)
    _acc(m_ref, mm, i)
    _acc(s_ref, ss, i)


def _p4_body(h_ref, a2_ref, c2_ref, o_ref):
    h1 = h_ref[...].astype(jnp.float32)
    h2 = _lrelu(jnp.dot(h1, a2_ref[...], preferred_element_type=jnp.float32)
                + c2_ref[...][:1])
    rb = h1.shape[0] // 8
    o_ref[...] = jnp.sum(h2.reshape(rb, 8, 128), axis=1)


def _big_spec(nrows):
    return pl.BlockSpec((nrows, 128), lambda i: (i, 0))


def _pt_spec(width):
    return pl.BlockSpec((_RB, width), lambda i: (i, 0))


def _const_spec(shape):
    return pl.BlockSpec(shape, lambda i: (0,) * len(shape))


def _tc_call(body, n_blocks, in_arrays, in_specs, out_shape, out_specs):
    return pl.pallas_call(
        body,
        grid=(n_blocks,),
        in_specs=in_specs,
        out_specs=out_specs,
        out_shape=out_shape,
        compiler_params=pltpu.CompilerParams(
            dimension_semantics=("arbitrary",)),
    )(*in_arrays)


def _fold(w, gamma, beta, mean_in, sec_in):
    mu = w @ mean_in
    e2 = jnp.einsum('oi,ij,oj->o', w, sec_in, w)
    var = e2 - mu * mu
    scale = gamma / jnp.sqrt(var + _EPS)
    a = scale[:, None] * w
    c = beta - scale * mu
    return a, c


def _bd4(a):
    return jnp.kron(jnp.eye(4, dtype=a.dtype), a.T)


def _diag_blocks_sum(m):
    return (m[0:32, 0:32] + m[32:64, 32:64] + m[64:96, 64:96]
            + m[96:128, 96:128])


def kernel(x, xyz, idx, W_xyz, g_xyz, b_xyz, W1, g1, b1, W2, g2, b2):
    b, c, n = x.shape
    k = idx.shape[-1]
    bn = b * n
    r = bn * k
    n_blocks = bn // _RB
    rows128 = _RB * 8

    x_t = jnp.transpose(x, (0, 2, 1)).reshape(bn, c)
    xyz_t = jnp.transpose(xyz, (0, 2, 1)).reshape(bn, 3)
    table_q = jnp.pad(xyz_t, ((0, 0), (0, 29)))
    ca = jnp.concatenate([
        jnp.zeros((bn, 3), jnp.float32), xyz_t,
        jnp.ones((bn, 1), jnp.float32),
        jnp.zeros((bn, 25), jnp.float32)], axis=1)
    idx_base = jnp.arange(b, dtype=idx.dtype)[:, None, None] * n
    idx2 = (idx + idx_base).reshape(r // 128, 128).astype(jnp.int32)

    qg = _sc_gather(table_q, idx2)
    xg = _sc_gather(x_t, idx2)
    xg128 = xg.reshape(r // 4, 128)
    qg128 = qg.reshape(r // 4, 128)

    m1, s1 = _tc_call(
        _p1_body, n_blocks,
        [qg128, ca],
        [_big_spec(rows128), _pt_spec(32)],
        [jax.ShapeDtypeStruct((128, 128), jnp.float32),
         jax.ShapeDtypeStruct((8, 128), jnp.float32)],
        [_const_spec((128, 128)), _const_spec((8, 128))],
    )
    m7 = _diag_blocks_sum(m1) / r
    egg, egc, ecc = m7[0:3, 0:3], m7[0:3, 3:6], m7[3:6, 3:6]
    mg, mc = m7[0:3, 6], m7[3:6, 6]
    ecg = egc.T
    mu9 = jnp.concatenate([mc, mg, mg - mc])
    row_c = jnp.concatenate([ecc, ecg, ecg - ecc], axis=1)
    row_g = jnp.concatenate([egc, egg, egg - egc], axis=1)
    row_d = row_g - row_c
    m9 = jnp.concatenate([row_c, row_g, row_d], axis=0)
    a_xyz, c_xyz = _fold(W_xyz, g_xyz, b_xyz, mu9, m9)
    p_mat = a_xyz[:, 0:3] - a_xyz[:, 6:9]
    q_mat = a_xyz[:, 3:6] + a_xyz[:, 6:9]
    qpad = jnp.zeros((32, 32), jnp.float32).at[:, 0:3].set(q_mat)
    qbdt = _bd4(qpad)
    uptt = (jnp.zeros((32, 32), jnp.float32)
            .at[:, 3:6].set(p_mat).at[:, 6].set(c_xyz)).T

    m2, s2, f16 = _tc_call(
        _p2_body, n_blocks,
        [xg128, qg128, x_t, ca, qbdt, uptt],
        [_big_spec(rows128), _big_spec(rows128), _pt_spec(32), _pt_spec(32),
         _const_spec((128, 128)), _const_spec((32, 32))],
        [jax.ShapeDtypeStruct((128, 128), jnp.float32),
         jax.ShapeDtypeStruct((8, 128), jnp.float32),
         jax.ShapeDtypeStruct((r // 4, 128), jnp.bfloat16)],
        [_const_spec((128, 128)), _const_spec((8, 128)),
         _big_spec(rows128)],
    )
    sec_f = _diag_blocks_sum(m2) / r
    mean_f = jnp.sum(s2.reshape(8, 4, 32), axis=(0, 1)) / r
    a1, c1 = _fold(W1, g1, b1, mean_f, sec_f)
    a1bdt = _bd4(a1)
    c1row = jnp.broadcast_to(jnp.tile(c1, 4)[None, :], (8, 128))

    m3, s3, h16 = _tc_call(
        _p3_body, n_blocks,
        [f16, a1bdt, c1row],
        [_big_spec(rows128),
         _const_spec((128, 128)), _const_spec((8, 128))],
        [jax.ShapeDtypeStruct((128, 128), jnp.float32),
         jax.ShapeDtypeStruct((8, 128), jnp.float32),
         jax.ShapeDtypeStruct((r // 4, 128), jnp.bfloat16)],
        [_const_spec((128, 128)), _const_spec((8, 128)),
         _big_spec(rows128)],
    )
    sec_h = _diag_blocks_sum(m3) / r
    mean_h = jnp.sum(s3.reshape(8, 4, 32), axis=(0, 1)) / r
    a2, c2 = _fold(W2, g2, b2, mean_h, sec_h)
    a2bdt = _bd4(a2)
    c2row = jnp.broadcast_to(jnp.tile(c2, 4)[None, :], (8, 128))

    osum = _tc_call(
        _p4_body, n_blocks,
        [h16, a2bdt, c2row],
        [_big_spec(rows128),
         _const_spec((128, 128)), _const_spec((8, 128))],
        jax.ShapeDtypeStruct((bn, 128), jnp.float32),
        _pt_spec(128),
    )
    out = jnp.sum(osum.reshape(b, n, 4, 32), axis=2) / k
    return jnp.transpose(out, (0, 2, 1))


# speedup vs baseline: 1.7118x; 1.0566x over previous
import functools

import jax
import jax.numpy as jnp
from jax import lax
from jax.experimental import pallas as pl
from jax.experimental.pallas import tpu as pltpu
from jax.experimental.pallas import tpu_sc as plsc

_SLOPE = 0.2
_EPS = 1e-5


_NW = 32
_SUB = 128
_NSUB = 16


def _sc_gather(table, idx2):
    dt = table.dtype
    n_rows_idx, lw = idx2.shape
    r_total = n_rows_idx * lw
    ch = _SUB * _NSUB
    per_w = r_total // _NW
    n_it = per_w // ch

    mesh = plsc.VectorSubcoreMesh(core_axis_name="c", subcore_axis_name="s")

    @functools.partial(
        pl.kernel,
        mesh=mesh,
        out_type=jax.ShapeDtypeStruct((r_total, 32), dt),
        scratch_types=[
            pltpu.VMEM((_NSUB, _SUB), jnp.int32),
            pltpu.VMEM((ch, 32), dt),
            pltpu.SemaphoreType.DMA,
        ],
        compiler_params=pltpu.CompilerParams(use_tc_tiling_on_sc=False),
    )
    def gather_kernel(t_hbm, idx_hbm, o_hbm, idx_v, r_v, sem):
        wid = lax.axis_index("s") * 2 + lax.axis_index("c")
        base_w = wid * per_w

        def body(it, carry):
            base = pl.multiple_of(base_w + it * ch, ch)
            idx_row = pl.multiple_of(base // lw, _NSUB)
            pltpu.sync_copy(idx_hbm.at[pl.ds(idx_row, _NSUB)], idx_v)
            copies = []
            for j in range(_NSUB):
                copies.append(pltpu.async_copy(
                    t_hbm.at[idx_v.at[j]],
                    r_v.at[pl.ds(j * _SUB, _SUB)], sem))
            for cp in copies:
                cp.wait()
            pltpu.sync_copy(r_v, o_hbm.at[pl.ds(base, ch)])
            return carry

        lax.fori_loop(0, n_it, body, 0)

    return gather_kernel(table, idx2)


_RB = 2048


def _lrelu(v):
    return jnp.where(v >= 0, v, _SLOPE * v)


def _feature(xg, qg, xc, ca, qbdt, uptt):
    rb = xc.shape[0]
    u_pt = jnp.dot(ca, uptt, preferred_element_type=jnp.float32)
    cent = u_pt - xc
    cent = jnp.concatenate([cent, cent, cent, cent], axis=1)
    cent = jnp.broadcast_to(cent[:, None, :], (rb, 8, 128))
    cent = cent.reshape(rb * 8, 128)
    qpart = jnp.dot(qg, qbdt, preferred_element_type=jnp.float32)
    return _lrelu(xg.astype(jnp.float32) + qpart + cent)


def _moments(v):
    mm = lax.dot_general(v, v, (((0,), (0,)), ((), ())),
                         preferred_element_type=jnp.float32)
    m = v.shape[0]
    ss = jnp.sum(v.reshape(8, m // 8, 128), axis=1)
    return mm, ss


def _acc(ref, val, step):
    @pl.when(step == 0)
    def _():
        ref[...] = val

    @pl.when(step != 0)
    def _():
        ref[...] += val


def _p1_body(qg_ref, ca_ref, m_ref, s_ref):
    i = pl.program_id(0)
    rb = ca_ref.shape[0]
    ca = ca_ref[...]
    cat = jnp.concatenate([ca, ca, ca, ca], axis=1)
    cat = jnp.broadcast_to(cat[:, None, :], (rb, 8, 128)).reshape(rb * 8, 128)
    w = qg_ref[...] + cat
    mm, ss = _moments(w)
    _acc(m_ref, mm, i)
    _acc(s_ref, ss, i)


def _p2_body(xg_ref, qg_ref, xc_ref, ca_ref, qbdt_ref, uptt_ref,
             m_ref, s_ref, f_ref):
    i = pl.program_id(0)
    f = _feature(xg_ref[...], qg_ref[...], xc_ref[...], ca_ref[...],
                 qbdt_ref[...], uptt_ref[...])
    f_ref[...] = f.astype(jnp.bfloat16)
    mm, ss = _moments(f)
    _acc(m_ref, mm, i)
    _acc(s_ref, ss, i)


def _p3_body(f_ref, a1_ref, c1_ref, m_ref, s_ref, h_ref):
    i = pl.program_id(0)
    f = f_ref[...].astype(jnp.float32)
    h1 = _lrelu(jnp.dot(f, a1_ref[...], preferred_element_type=jnp.float32)
                + c1_ref[...][:1])
    h_ref[...] = h1.astype(jnp.bfloat16)
    mm, ss = _moments(h1)
    _acc(m_ref, mm, i)
    _acc(s_ref, ss, i)


def _p4_body(h_ref, a2_ref, c2_ref, o_ref):
    h1 = h_ref[...].astype(jnp.float32)
    h2 = _lrelu(jnp.dot(h1, a2_ref[...], preferred_element_type=jnp.float32)
                + c2_ref[...][:1])
    rb = h1.shape[0] // 8
    o_ref[...] = jnp.sum(h2.reshape(rb, 8, 128), axis=1)


def _big_spec(nrows):
    return pl.BlockSpec((nrows, 128), lambda i: (i, 0))


def _pt_spec(width):
    return pl.BlockSpec((_RB, width), lambda i: (i, 0))


def _const_spec(shape):
    return pl.BlockSpec(shape, lambda i: (0,) * len(shape))


def _tc_call(body, n_blocks, in_arrays, in_specs, out_shape, out_specs):
    return pl.pallas_call(
        body,
        grid=(n_blocks,),
        in_specs=in_specs,
        out_specs=out_specs,
        out_shape=out_shape,
        compiler_params=pltpu.CompilerParams(
            dimension_semantics=("arbitrary",)),
    )(*in_arrays)


def _fold(w, gamma, beta, mean_in, sec_in):
    mu = w @ mean_in
    e2 = jnp.einsum('oi,ij,oj->o', w, sec_in, w)
    var = e2 - mu * mu
    scale = gamma / jnp.sqrt(var + _EPS)
    a = scale[:, None] * w
    c = beta - scale * mu
    return a, c


def _bd4(a):
    return jnp.kron(jnp.eye(4, dtype=a.dtype), a.T)


def _diag_blocks_sum(m):
    return (m[0:32, 0:32] + m[32:64, 32:64] + m[64:96, 64:96]
            + m[96:128, 96:128])


def kernel(x, xyz, idx, W_xyz, g_xyz, b_xyz, W1, g1, b1, W2, g2, b2):
    b, c, n = x.shape
    k = idx.shape[-1]
    bn = b * n
    r = bn * k
    n_blocks = bn // _RB
    rows128 = _RB * 8

    x_t = jnp.transpose(x, (0, 2, 1)).reshape(bn, c)
    xyz_t = jnp.transpose(xyz, (0, 2, 1)).reshape(bn, 3)
    table_q = jnp.pad(xyz_t, ((0, 0), (0, 29)))
    ca = jnp.concatenate([
        jnp.zeros((bn, 3), jnp.float32), xyz_t,
        jnp.ones((bn, 1), jnp.float32),
        jnp.zeros((bn, 25), jnp.float32)], axis=1)
    idx_base = jnp.arange(b, dtype=idx.dtype)[:, None, None] * n
    idx2 = (idx + idx_base).reshape(r // 128, 128).astype(jnp.int32)

    qg = _sc_gather(table_q, idx2)
    xg = _sc_gather(x_t, idx2)
    xg128 = xg.reshape(r // 4, 128)
    qg128 = qg.reshape(r // 4, 128)

    m1, s1 = _tc_call(
        _p1_body, n_blocks,
        [qg128, ca],
        [_big_spec(rows128), _pt_spec(32)],
        [jax.ShapeDtypeStruct((128, 128), jnp.float32),
         jax.ShapeDtypeStruct((8, 128), jnp.float32)],
        [_const_spec((128, 128)), _const_spec((8, 128))],
    )
    m7 = _diag_blocks_sum(m1) / r
    egg, egc, ecc = m7[0:3, 0:3], m7[0:3, 3:6], m7[3:6, 3:6]
    mg, mc = m7[0:3, 6], m7[3:6, 6]
    ecg = egc.T
    mu9 = jnp.concatenate([mc, mg, mg - mc])
    row_c = jnp.concatenate([ecc, ecg, ecg - ecc], axis=1)
    row_g = jnp.concatenate([egc, egg, egg - egc], axis=1)
    row_d = row_g - row_c
    m9 = jnp.concatenate([row_c, row_g, row_d], axis=0)
    a_xyz, c_xyz = _fold(W_xyz, g_xyz, b_xyz, mu9, m9)
    p_mat = a_xyz[:, 0:3] - a_xyz[:, 6:9]
    q_mat = a_xyz[:, 3:6] + a_xyz[:, 6:9]
    qpad = jnp.zeros((32, 32), jnp.float32).at[:, 0:3].set(q_mat)
    qbdt = _bd4(qpad)
    uptt = (jnp.zeros((32, 32), jnp.float32)
            .at[:, 3:6].set(p_mat).at[:, 6].set(c_xyz)).T

    m2, s2, f16 = _tc_call(
        _p2_body, n_blocks,
        [xg128, qg128, x_t, ca, qbdt, uptt],
        [_big_spec(rows128), _big_spec(rows128), _pt_spec(32), _pt_spec(32),
         _const_spec((128, 128)), _const_spec((32, 32))],
        [jax.ShapeDtypeStruct((128, 128), jnp.float32),
         jax.ShapeDtypeStruct((8, 128), jnp.float32),
         jax.ShapeDtypeStruct((r // 4, 128), jnp.bfloat16)],
        [_const_spec((128, 128)), _const_spec((8, 128)),
         _big_spec(rows128)],
    )
    sec_f = _diag_blocks_sum(m2) / r
    mean_f = jnp.sum(s2.reshape(8, 4, 32), axis=(0, 1)) / r
    a1, c1 = _fold(W1, g1, b1, mean_f, sec_f)
    a1bdt = _bd4(a1)
    c1row = jnp.broadcast_to(jnp.tile(c1, 4)[None, :], (8, 128))

    m3, s3, h16 = _tc_call(
        _p3_body, n_blocks,
        [f16, a1bdt, c1row],
        [_big_spec(rows128),
         _const_spec((128, 128)), _const_spec((8, 128))],
        [jax.ShapeDtypeStruct((128, 128), jnp.float32),
         jax.ShapeDtypeStruct((8, 128), jnp.float32),
         jax.ShapeDtypeStruct((r // 4, 128), jnp.bfloat16)],
        [_const_spec((128, 128)), _const_spec((8, 128)),
         _big_spec(rows128)],
    )
    sec_h = _diag_blocks_sum(m3) / r
    mean_h = jnp.sum(s3.reshape(8, 4, 32), axis=(0, 1)) / r
    a2, c2 = _fold(W2, g2, b2, mean_h, sec_h)
    a2bdt = _bd4(a2)
    c2row = jnp.broadcast_to(jnp.tile(c2, 4)[None, :], (8, 128))

    osum = _tc_call(
        _p4_body, n_blocks,
        [h16, a2bdt, c2row],
        [_big_spec(rows128),
         _const_spec((128, 128)), _const_spec((8, 128))],
        jax.ShapeDtypeStruct((bn, 128), jnp.float32),
        _pt_spec(128),
    )
    out = jnp.sum(osum.reshape(b, n, 4, 32), axis=2) / k
    return jnp.transpose(out, (0, 2, 1))
